# baseline (speedup 1.0000x reference)
import jax
import jax.numpy as jnp
from jax.experimental import pallas as pl


def _transpose_kernel(f_ref, o_ref):
    o_ref[0] = f_ref[0].T


def kernel(xyz, xyz_fp, features, features_fp, W, b):
    B, C, N = features.shape
    out = pl.pallas_call(
        _transpose_kernel,
        grid=(B,),
        in_specs=[pl.BlockSpec((1, C, N), lambda i: (i, 0, 0))],
        out_specs=pl.BlockSpec((1, N, C), lambda i: (i, 0, 0)),
        out_shape=jax.ShapeDtypeStruct((B, N, C), features.dtype),
    )(features)
    return out
